# baseline (speedup 1.0000x reference)
import functools

import jax
import jax.numpy as jnp
from jax import lax
from jax.experimental import pallas as pl
from jax.experimental.pallas import tpu as pltpu
from jax.experimental.pallas import tpu_sc as plsc

LANES = 16


def _make_pool_kernel(B, S, D, NC, NS):
    NW = NC * NS
    bw = B // NW
    CB = 2
    CR = CB * S
    NCH = bw // CB
    assert B % NW == 0 and bw % CB == 0 and CR <= 128 and D % LANES == 0
    assert NCH % 2 == 0
    RV = D // LANES

    mesh = plsc.VectorSubcoreMesh(core_axis_name="c", subcore_axis_name="s")

    @functools.partial(
        pl.kernel,
        mesh=mesh,
        compiler_params=pltpu.CompilerParams(use_tc_tiling_on_sc=False),
        out_type=jax.ShapeDtypeStruct((B, D), jnp.float32),
        scratch_types=[
            pltpu.VMEM((NCH, CR), jnp.int32),
            pltpu.VMEM((CR, D), jnp.float32),
            pltpu.VMEM((CR, D), jnp.float32),
            pltpu.VMEM((bw, D), jnp.float32),
            pltpu.SemaphoreType.DMA,
            pltpu.SemaphoreType.DMA,
        ],
    )
    def pool(ids_hbm, table_hbm, out_hbm, idx_v, rows0, rows1, pooled_v, sem0, sem1):
        wid = lax.axis_index("s") * NC + lax.axis_index("c")
        pltpu.sync_copy(ids_hbm.at[wid], idx_v)
        scale = jnp.float32(1.0 / S)

        def gather_start(c, buf, sem):
            pltpu.make_async_copy(table_hbm.at[idx_v.at[c]], buf, sem).start()

        def gather_wait(buf, sem):
            pltpu.make_async_copy(table_hbm.at[idx_v.at[0]], buf, sem).wait()

        def accum(c, buf):
            for bloc in range(CB):
                def seq_body(s, acc, _bloc=bloc):
                    row = _bloc * S + s
                    return tuple(acc[r] + buf[row, pl.ds(LANES * r, LANES)]
                                 for r in range(RV))
                acc = lax.fori_loop(
                    0, S, seq_body,
                    tuple(jnp.zeros((LANES,), jnp.float32) for _ in range(RV)))
                out_row = c * CB + bloc
                for r in range(RV):
                    pooled_v[out_row, pl.ds(LANES * r, LANES)] = acc[r] * scale

        gather_start(0, rows0, sem0)

        def chunk_pair(i, carry):
            c = 2 * i
            gather_start(c + 1, rows1, sem1)
            gather_wait(rows0, sem0)
            accum(c, rows0)

            @pl.when(c + 2 < NCH)
            def _():
                gather_start(c + 2, rows0, sem0)

            gather_wait(rows1, sem1)
            accum(c + 1, rows1)
            return carry

        lax.fori_loop(0, NCH // 2, chunk_pair, 0)
        pltpu.sync_copy(pooled_v, out_hbm.at[pl.ds(wid * bw, bw)])

    return pool


def _classifier(pooled, W, b):
    B, D = pooled.shape
    C = W.shape[1]
    BLK = 512

    def mm_body(p_ref, w_ref, b_ref, o_ref):
        o_ref[...] = (
            jnp.dot(p_ref[...], w_ref[...], preferred_element_type=jnp.float32)
            + b_ref[...]
        )

    return pl.pallas_call(
        mm_body,
        grid=(B // BLK,),
        in_specs=[
            pl.BlockSpec((BLK, D), lambda i: (i, 0)),
            pl.BlockSpec((D, C), lambda i: (0, 0)),
            pl.BlockSpec((1, C), lambda i: (0, 0)),
        ],
        out_specs=pl.BlockSpec((BLK, C), lambda i: (i, 0)),
        out_shape=jax.ShapeDtypeStruct((B, C), jnp.float32),
    )(pooled, W, b.reshape(1, C))


def kernel(input_ids, table, W, b):
    B, S = input_ids.shape
    V, D = table.shape
    info = plsc.get_sparse_core_info()
    NC, NS = info.num_cores, info.num_subcores
    NW = NC * NS
    CB = 2
    ids = input_ids.astype(jnp.int32).reshape(NW, (B // NW) // CB, CB * S)
    pooled = _make_pool_kernel(B, S, D, NC, NS)(ids, table)
    return _classifier(pooled, W, b)
